# Initial kernel scaffold; baseline (speedup 1.0000x reference)
import jax
import jax.numpy as jnp
from jax.experimental import pallas as pl


def kernel(x, in_feat_size, expert_weights, expert_indices, x_final, hidden_W, hidden_b, out_W, out_b, res_W, res_b, ln_w):
    raise NotImplementedError("write your pallas kernel here")


# speedup vs baseline: 1.3775x; 1.3775x over previous
import jax
import jax.numpy as jnp
from jax.experimental import pallas as pl

IN_DIM_LS = (128, 256, 384, 512)
H_DIM = 1024
OUT_DIM = 768
TOP_K = 2
TOKENS = 2048
MAX_FEAT = max(IN_DIM_LS)
N_EXPERTS = len(IN_DIM_LS)
TILE = 256
N_TILES = TOKENS // TILE


def _block_kernel(ew_ref, idx_ref, xf_ref, hW_ref, hb_ref, oW_ref, ob_ref,
                  rW_ref, rb_ref, ln_ref, out_ref):
    i = pl.program_id(0)
    ew = ew_ref[pl.ds(i * TILE, TILE), :]
    idx = idx_ref[pl.ds(i * TILE, TILE), :]
    is_real = (idx < N_EXPERTS).astype(jnp.float32)
    real_sum = jnp.sum(ew * is_real, axis=-1, keepdims=True)
    real_sum = jnp.where(real_sum == 0.0, 1.0, real_sum)
    wnorm = ew / real_sum

    col = jax.lax.broadcasted_iota(jnp.int32, (1, 2 * MAX_FEAT), 1) % MAX_FEAT

    hid = jnp.zeros((TILE, H_DIM), dtype=jnp.float32)
    res = jnp.zeros((TILE, OUT_DIM), dtype=jnp.float32)
    coeffs = []
    for e in range(N_EXPERTS):
        ce = jnp.sum(wnorm * (idx == e).astype(jnp.float32), axis=-1,
                     keepdims=True)
        coeffs.append(ce)
        mask_e = (col < IN_DIM_LS[e]).astype(jnp.float32)
        z_e = (xf_ref[e].astype(jnp.float32) * ce * mask_e).astype(jnp.bfloat16)
        hid = hid + jax.lax.dot_general(
            z_e, hW_ref[e], (((1,), (1,)), ((), ())),
            preferred_element_type=jnp.float32)
        res = res + jax.lax.dot_general(
            z_e, rW_ref[e], (((1,), (1,)), ((), ())),
            preferred_element_type=jnp.float32)
    coeff = jnp.concatenate(coeffs, axis=-1)
    hid = hid + jax.lax.dot_general(
        coeff, hb_ref[:], (((1,), (0,)), ((), ())),
        preferred_element_type=jnp.float32)
    res = res + jax.lax.dot_general(
        coeff, rb_ref[:], (((1,), (0,)), ((), ())),
        preferred_element_type=jnp.float32)
    hid = jax.nn.gelu(hid).astype(jnp.bfloat16)
    out = jax.lax.dot_general(
        hid, oW_ref[:], (((1,), (1,)), ((), ())),
        preferred_element_type=jnp.float32) + ob_ref[:] + res
    var = jnp.mean(jnp.square(out), axis=-1, keepdims=True)
    out_ref[:] = out * jax.lax.rsqrt(var + 1e-6) * ln_ref[:]


def kernel(x, in_feat_size, expert_weights, expert_indices, x_final,
           hidden_W, hidden_b, out_W, out_b, res_W, res_b, ln_w):
    del x, in_feat_size
    xf = x_final.astype(jnp.bfloat16)
    hW = hidden_W.astype(jnp.bfloat16)
    rW = res_W.astype(jnp.bfloat16)
    oW = out_W.astype(jnp.bfloat16)
    ob = out_b.reshape(1, OUT_DIM)
    ln = ln_w.reshape(1, OUT_DIM)
    grid = (N_TILES,)
    full = lambda i: (0, 0)
    full3 = lambda i: (0, 0, 0)
    out = pl.pallas_call(
        _block_kernel,
        grid=grid,
        in_specs=[
            pl.BlockSpec((TOKENS, TOP_K), full),
            pl.BlockSpec((TOKENS, TOP_K), full),
            pl.BlockSpec((N_EXPERTS, TILE, 2 * MAX_FEAT),
                         lambda i: (0, i, 0)),
            pl.BlockSpec((N_EXPERTS, H_DIM, 2 * MAX_FEAT), full3),
            pl.BlockSpec((N_EXPERTS, H_DIM), full),
            pl.BlockSpec((OUT_DIM, H_DIM), full),
            pl.BlockSpec((1, OUT_DIM), full),
            pl.BlockSpec((N_EXPERTS, OUT_DIM, 2 * MAX_FEAT), full3),
            pl.BlockSpec((N_EXPERTS, OUT_DIM), full),
            pl.BlockSpec((1, OUT_DIM), full),
        ],
        out_specs=pl.BlockSpec((TILE, OUT_DIM), lambda i: (i, 0)),
        out_shape=jax.ShapeDtypeStruct((TOKENS, OUT_DIM), jnp.float32),
    )(expert_weights, expert_indices, xf, hW, hidden_b,
      oW, ob, rW, res_b, ln)
    return out


# speedup vs baseline: 1.5643x; 1.1356x over previous
import jax
import jax.numpy as jnp
import numpy as np
from jax.experimental import pallas as pl

IN_DIM_LS = (128, 256, 384, 512)
H_DIM = 1024
OUT_DIM = 768
TOP_K = 2
TOKENS = 2048
MAX_FEAT = max(IN_DIM_LS)
N_EXPERTS = len(IN_DIM_LS)
COMB = H_DIM + OUT_DIM
TILE = 512
N_TILES = TOKENS // TILE


def _block_kernel(ew_ref, idx_ref, xf_ref, W_ref, hb_ref, oW_ref, ob_ref,
                  rb_ref, ln_ref, out_ref):
    i = pl.program_id(0)
    ew = ew_ref[pl.ds(i * TILE, TILE), :]
    idx = idx_ref[pl.ds(i * TILE, TILE), :]
    is_real = (idx < N_EXPERTS).astype(jnp.float32)
    real_sum = jnp.sum(ew * is_real, axis=-1, keepdims=True)
    real_sum = jnp.where(real_sum == 0.0, 1.0, real_sum)
    wnorm = ew / real_sum

    hid = jnp.zeros((TILE, H_DIM), dtype=jnp.float32)
    res = jnp.zeros((TILE, OUT_DIM), dtype=jnp.float32)
    coeffs = []
    for e in range(N_EXPERTS):
        ce = jnp.sum(wnorm * (idx == e).astype(jnp.float32), axis=-1,
                     keepdims=True)
        coeffs.append(ce)
        p_e = jax.lax.dot_general(
            xf_ref[e], W_ref[e], (((1,), (1,)), ((), ())),
            preferred_element_type=jnp.float32)
        hid = hid + p_e[:, :H_DIM] * ce
        res = res + p_e[:, H_DIM:] * ce
    coeff = jnp.concatenate(coeffs, axis=-1)
    hid = hid + jax.lax.dot_general(
        coeff, hb_ref[:], (((1,), (0,)), ((), ())),
        preferred_element_type=jnp.float32)
    res = res + jax.lax.dot_general(
        coeff, rb_ref[:], (((1,), (0,)), ((), ())),
        preferred_element_type=jnp.float32)
    hid = jax.nn.gelu(hid).astype(jnp.bfloat16)
    out = jax.lax.dot_general(
        hid, oW_ref[:], (((1,), (1,)), ((), ())),
        preferred_element_type=jnp.float32) + ob_ref[:] + res
    var = jnp.mean(jnp.square(out), axis=-1, keepdims=True)
    out_ref[:] = out * jax.lax.rsqrt(var + 1e-6) * ln_ref[:]


def kernel(x, in_feat_size, expert_weights, expert_indices, x_final,
           hidden_W, hidden_b, out_W, out_b, res_W, res_b, ln_w):
    del x, in_feat_size
    sizes = np.asarray(IN_DIM_LS)
    col = np.arange(MAX_FEAT)
    m = (col[None, :] < sizes[:, None]).astype(np.float32)
    mask = jnp.asarray(np.concatenate([m, m], axis=-1))
    Wcat = jnp.concatenate([hidden_W, res_W], axis=1)
    Wcat = (Wcat * mask[:, None, :]).astype(jnp.bfloat16)
    xf = x_final.astype(jnp.bfloat16)
    oW = out_W.astype(jnp.bfloat16)
    ob = out_b.reshape(1, OUT_DIM)
    ln = ln_w.reshape(1, OUT_DIM)
    grid = (N_TILES,)
    full = lambda i: (0, 0)
    full3 = lambda i: (0, 0, 0)
    out = pl.pallas_call(
        _block_kernel,
        grid=grid,
        in_specs=[
            pl.BlockSpec((TOKENS, TOP_K), full),
            pl.BlockSpec((TOKENS, TOP_K), full),
            pl.BlockSpec((N_EXPERTS, TILE, 2 * MAX_FEAT),
                         lambda i: (0, i, 0)),
            pl.BlockSpec((N_EXPERTS, COMB, 2 * MAX_FEAT), full3),
            pl.BlockSpec((N_EXPERTS, H_DIM), full),
            pl.BlockSpec((OUT_DIM, H_DIM), full),
            pl.BlockSpec((1, OUT_DIM), full),
            pl.BlockSpec((N_EXPERTS, OUT_DIM), full),
            pl.BlockSpec((1, OUT_DIM), full),
        ],
        out_specs=pl.BlockSpec((TILE, OUT_DIM), lambda i: (i, 0)),
        out_shape=jax.ShapeDtypeStruct((TOKENS, OUT_DIM), jnp.float32),
    )(expert_weights, expert_indices, xf, Wcat, hidden_b,
      oW, ob, res_b, ln)
    return out


# speedup vs baseline: 2.7108x; 1.7330x over previous
import jax
import jax.numpy as jnp
from jax.experimental import pallas as pl
from jax.experimental.pallas import tpu as pltpu

IN_DIM_LS = (128, 256, 384, 512)
H_DIM = 1024
OUT_DIM = 768
TOP_K = 2
TOKENS = 2048
MAX_FEAT = max(IN_DIM_LS)
N_EXPERTS = len(IN_DIM_LS)
TILE = 256
N_TILES = TOKENS // TILE

_SLICES = []
_off = 0
for _s in IN_DIM_LS:
    for _h in range(2):
        _SLICES.append((_h * MAX_FEAT, _s, _off))
        _off += _s
_SLICES = tuple(_SLICES)
PACKED = _off


def _block_kernel(ew_ref, idx_ref, xf_ref, hW_ref, rW_ref, hb_ref, oW_ref,
                  ob_ref, rb_ref, ln_ref, out_ref, hW16, rW16):
    i = pl.program_id(0)

    @pl.when(i == 0)
    def _cast_weights():
        for e in range(N_EXPERTS):
            for h in range(2):
                a, s, off = _SLICES[2 * e + h]
                hW16[:, off:off + s] = hW_ref[e, :, a:a + s].astype(jnp.bfloat16)
                rW16[:, off:off + s] = rW_ref[e, :, a:a + s].astype(jnp.bfloat16)

    ew = ew_ref[:, pl.ds(i * TILE, TILE)]
    idx = idx_ref[:, pl.ds(i * TILE, TILE)]
    is_real = (idx < N_EXPERTS).astype(jnp.float32)
    real_sum = jnp.sum(ew * is_real, axis=0, keepdims=True)
    real_sum = jnp.where(real_sum == 0.0, 1.0, real_sum)
    wnorm = ew / real_sum

    hid = jnp.zeros((TILE, H_DIM), dtype=jnp.float32)
    res = jnp.zeros((TILE, OUT_DIM), dtype=jnp.float32)
    ces = []
    dims = (((1,), (1,)), ((), ()))
    for e in range(N_EXPERTS):
        ce = jnp.sum(wnorm * (idx == e).astype(jnp.float32), axis=0)
        ce = ce[:, None]
        ces.append(ce)
        he = jnp.zeros((TILE, H_DIM), dtype=jnp.float32)
        re = jnp.zeros((TILE, OUT_DIM), dtype=jnp.float32)
        for h in range(2):
            a, s, off = _SLICES[2 * e + h]
            xe = xf_ref[e][:, a:a + s].astype(jnp.bfloat16)
            he = he + jax.lax.dot_general(
                xe, hW16[:, off:off + s], dims,
                preferred_element_type=jnp.float32)
            re = re + jax.lax.dot_general(
                xe, rW16[:, off:off + s], dims,
                preferred_element_type=jnp.float32)
        hid = hid + he * ce
        res = res + re * ce
    coeff = jnp.concatenate(ces, axis=-1)
    hid = hid + jax.lax.dot_general(
        coeff, hb_ref[:], (((1,), (0,)), ((), ())),
        preferred_element_type=jnp.float32)
    res = res + jax.lax.dot_general(
        coeff, rb_ref[:], (((1,), (0,)), ((), ())),
        preferred_element_type=jnp.float32)
    hid = jax.nn.gelu(hid).astype(jnp.bfloat16)
    out = jax.lax.dot_general(
        hid, oW_ref[:], dims,
        preferred_element_type=jnp.float32) + ob_ref[:] + res
    var = jnp.mean(jnp.square(out), axis=-1, keepdims=True)
    out_ref[:] = out * jax.lax.rsqrt(var + 1e-6) * ln_ref[:]


def kernel(x, in_feat_size, expert_weights, expert_indices, x_final,
           hidden_W, hidden_b, out_W, out_b, res_W, res_b, ln_w):
    del x, in_feat_size
    ew_t = expert_weights.T
    idx_t = expert_indices.T
    oW16 = out_W.astype(jnp.bfloat16)
    ob = out_b.reshape(1, OUT_DIM)
    ln = ln_w.reshape(1, OUT_DIM)
    grid = (N_TILES,)
    full = lambda i: (0, 0)
    full3 = lambda i: (0, 0, 0)
    out = pl.pallas_call(
        _block_kernel,
        grid=grid,
        in_specs=[
            pl.BlockSpec((TOP_K, TOKENS), full),
            pl.BlockSpec((TOP_K, TOKENS), full),
            pl.BlockSpec((N_EXPERTS, TILE, 2 * MAX_FEAT),
                         lambda i: (0, i, 0)),
            pl.BlockSpec((N_EXPERTS, H_DIM, 2 * MAX_FEAT), full3),
            pl.BlockSpec((N_EXPERTS, OUT_DIM, 2 * MAX_FEAT), full3),
            pl.BlockSpec((N_EXPERTS, H_DIM), full),
            pl.BlockSpec((OUT_DIM, H_DIM), full),
            pl.BlockSpec((1, OUT_DIM), full),
            pl.BlockSpec((N_EXPERTS, OUT_DIM), full),
            pl.BlockSpec((1, OUT_DIM), full),
        ],
        out_specs=pl.BlockSpec((TILE, OUT_DIM), lambda i: (i, 0)),
        out_shape=jax.ShapeDtypeStruct((TOKENS, OUT_DIM), jnp.float32),
        scratch_shapes=[
            pltpu.VMEM((H_DIM, PACKED), jnp.bfloat16),
            pltpu.VMEM((OUT_DIM, PACKED), jnp.bfloat16),
        ],
    )(ew_t, idx_t, x_final, hidden_W, res_W, hidden_b,
      oW16, ob, res_b, ln)
    return out
